# baseline (speedup 1.0000x reference)
import functools

import jax
import jax.numpy as jnp
from jax import lax
from jax.experimental import pallas as pl
from jax.experimental.pallas import tpu as pltpu
from jax.experimental.pallas import tpu_sc as plsc

_L = 16
_CHUNK = 32
_NCHUNK = 7
_PADLEN = _CHUNK * _NCHUNK
_NGRP = 4
_VOCAB = 100000
_C = 64
_SEQLEN = 200


def _mc_body(seq_hbm, delta_hbm, out_hbm,
             seq_v, idx_v, gath_v, tot_v, totbuf_v, out_v, rowidx_v,
             shared, sem):
    c = lax.axis_index("c")
    s = lax.axis_index("s")
    active = s < 2 * _NCHUNK
    g_local = s // _NCHUNK
    j = s % _NCHUNK
    g = c * 2 + g_local

    @pl.when(active)
    def _gather_and_scan():
        pltpu.sync_copy(seq_hbm, seq_v)
        lane = lax.iota(jnp.int32, _L)
        base = (g * _L + lane) * _VOCAB
        for t in range(_CHUNK):
            pos = jnp.full((_L,), j * _CHUNK + t, jnp.int32)
            sv = plsc.load_gather(seq_v, [pos])
            idx_v[pl.ds(t * _L, _L)] = sv + base
        copies = [
            pltpu.async_copy(
                delta_hbm.at[idx_v.at[pl.ds(r * 128, 128)]],
                gath_v.at[pl.ds(r * 128, 128)],
                sem,
            )
            for r in range(_CHUNK * _L // 128)
        ]
        for cp in copies:
            cp.wait()
        acc = jnp.zeros((_L,), jnp.float32)
        for t in range(_CHUNK):
            acc = acc + gath_v[pl.ds(t * _L, _L)]
            out_v[t] = acc
        tot_v[...] = acc
        pltpu.sync_copy(tot_v, shared.at[s])

    plsc.subcore_barrier()

    @pl.when(active)
    def _prefix_and_write():
        pltpu.sync_copy(shared, totbuf_v)
        offset = jnp.zeros((_L,), jnp.float32)
        for i in range(_NCHUNK - 1):
            row = totbuf_v[g_local * _NCHUNK + i]
            offset = offset + jnp.where(i < j, row, jnp.zeros((_L,), jnp.float32))
        for t in range(_CHUNK):
            out_v[t] = out_v[t] + offset
        lane = lax.iota(jnp.int32, _L)
        for b in range(_CHUNK // _L):
            t0 = j * _CHUNK + b * _L
            rowidx_v[pl.ds(b * _L, _L)] = (t0 + lane) * _NGRP + g
        pltpu.async_copy(out_v, out_hbm.at[rowidx_v], sem).wait()


@functools.partial(jax.jit, static_argnames=())
def _mc(seq_pad, delta_flat):
    mesh = plsc.VectorSubcoreMesh(core_axis_name="c", subcore_axis_name="s")
    f = functools.partial(
        pl.kernel,
        out_type=jax.ShapeDtypeStruct((_PADLEN * _NGRP, _L), jnp.float32),
        mesh=mesh,
        compiler_params=pltpu.CompilerParams(
            needs_layout_passes=False, use_tc_tiling_on_sc=False),
        scratch_types=[
            pltpu.VMEM((_PADLEN,), jnp.int32),
            pltpu.VMEM((_CHUNK * _L,), jnp.int32),
            pltpu.VMEM((_CHUNK * _L,), jnp.float32),
            pltpu.VMEM((_L,), jnp.float32),
            pltpu.VMEM((2 * _L, _L), jnp.float32),
            pltpu.VMEM((_CHUNK, _L), jnp.float32),
            pltpu.VMEM((_CHUNK,), jnp.int32),
            pltpu.VMEM_SHARED((2 * _L, _L), jnp.float32),
            pltpu.SemaphoreType.DMA,
        ],
    )(_mc_body)
    return f(seq_pad, delta_flat)


def kernel(input_seq, delta):
    seq = input_seq.astype(jnp.int32)
    seq_pad = jnp.concatenate([seq, jnp.zeros((_PADLEN - _SEQLEN,), jnp.int32)])
    out = _mc(seq_pad, delta.reshape(-1))
    return out.reshape(_PADLEN, _C)[:_SEQLEN]


# speedup vs baseline: 2.0454x; 2.0454x over previous
import functools

import jax
import jax.numpy as jnp
from jax import lax
from jax.experimental import pallas as pl
from jax.experimental.pallas import tpu as pltpu
from jax.experimental.pallas import tpu_sc as plsc

_L = 16
_CHUNK = 16
_NCHUNK = 14
_PADLEN = _CHUNK * _NCHUNK
_VOCAB = 100000
_C = 64
_HALF = 32
_SEQLEN = 200
_NKT = _HALF // 8


def _mc_body(seq_hbm, delta_hbm, out0_hbm, out1_hbm,
             seq_v, tiles_v, out_v, tot_v, totbuf_v, shared, sem):
    c = lax.axis_index("c")
    s = lax.axis_index("s")
    active = s < _NCHUNK
    j = s

    @pl.when(active)
    def _gather_and_scan():
        pltpu.sync_copy(seq_hbm, seq_v.at[pl.ds(0, _SEQLEN)])
        sv16 = seq_v[pl.ds(pl.multiple_of(j * _CHUNK, _CHUNK), _CHUNK)]
        sv16 = jnp.clip(sv16, 0, _VOCAB - 1)
        copies = []
        for p in range(_CHUNK):
            v = sv16[p]
            col0 = pl.multiple_of((v >> 7) * 128, 128)
            for k in range(_NKT):
                copies.append(pltpu.async_copy(
                    delta_hbm.at[pl.ds((c * _NKT + k) * 8, 8),
                                 pl.ds(col0, 128)],
                    tiles_v.at[pl.ds((p * _NKT + k) * 8, 8), :],
                    sem,
                ))
        for cp in copies:
            cp.wait()
        lane = lax.iota(jnp.int32, _L)
        acc0 = jnp.zeros((_L,), jnp.float32)
        acc1 = jnp.zeros((_L,), jnp.float32)
        voff16 = sv16 & 127
        for p in range(_CHUNK):
            voff = jnp.full((_L,), voff16[p], jnp.int32)
            r0 = jnp.full((_L,), p * _HALF, jnp.int32) + lane
            acc0 = acc0 + plsc.load_gather(tiles_v, [r0, voff])
            acc1 = acc1 + plsc.load_gather(tiles_v, [r0 + _L, voff])
            out_v[p, pl.ds(0, _L)] = acc0
            out_v[p, pl.ds(_L, _L)] = acc1
        tot_v[pl.ds(0, _L)] = acc0
        tot_v[pl.ds(_L, _L)] = acc1
        pltpu.sync_copy(tot_v, shared.at[c * _L + s, pl.ds(0, _HALF)])

    plsc.subcore_barrier()

    @pl.when(active)
    def _prefix_and_write():
        pltpu.sync_copy(
            shared.at[pl.ds(pl.multiple_of(c * _L, _L), _L), :], totbuf_v)
        off0 = jnp.zeros((_L,), jnp.float32)
        off1 = jnp.zeros((_L,), jnp.float32)
        zero = jnp.zeros((_L,), jnp.float32)
        for i in range(_NCHUNK - 1):
            sel = i < j
            off0 = off0 + jnp.where(sel, totbuf_v[i, pl.ds(0, _L)], zero)
            off1 = off1 + jnp.where(sel, totbuf_v[i, pl.ds(_L, _L)], zero)
        for p in range(_CHUNK):
            out_v[p, pl.ds(0, _L)] = out_v[p, pl.ds(0, _L)] + off0
            out_v[p, pl.ds(_L, _L)] = out_v[p, pl.ds(_L, _L)] + off1

        @pl.when(c == 0)
        def _w0():
            pltpu.sync_copy(out_v, out0_hbm.at[pl.ds(j * _CHUNK, _CHUNK), :])

        @pl.when(c == 1)
        def _w1():
            pltpu.sync_copy(out_v, out1_hbm.at[pl.ds(j * _CHUNK, _CHUNK), :])


@jax.jit
def _mc(seq, delta):
    mesh = plsc.VectorSubcoreMesh(core_axis_name="c", subcore_axis_name="s")
    f = functools.partial(
        pl.kernel,
        out_type=[
            jax.ShapeDtypeStruct((_PADLEN, _HALF), jnp.float32),
            jax.ShapeDtypeStruct((_PADLEN, _HALF), jnp.float32),
        ],
        mesh=mesh,
        compiler_params=pltpu.CompilerParams(
            needs_layout_passes=False, use_tc_tiling_on_sc=True),
        scratch_types=[
            pltpu.VMEM((_PADLEN,), jnp.int32),
            pltpu.VMEM((_CHUNK * _NKT * 8, 128), jnp.float32),
            pltpu.VMEM((_CHUNK, _HALF), jnp.float32),
            pltpu.VMEM((_HALF,), jnp.float32),
            pltpu.VMEM((_L, 128), jnp.float32),
            pltpu.VMEM_SHARED((2 * _L, 128), jnp.float32),
            pltpu.SemaphoreType.DMA,
        ],
    )(_mc_body)
    return f(seq, delta)


def kernel(input_seq, delta):
    out0, out1 = _mc(input_seq.astype(jnp.int32), delta)
    return jnp.concatenate([out0, out1], axis=1)[:_SEQLEN]


# speedup vs baseline: 2.1158x; 1.0344x over previous
import functools

import jax
import jax.numpy as jnp
from jax import lax
from jax.experimental import pallas as pl
from jax.experimental.pallas import tpu as pltpu
from jax.experimental.pallas import tpu_sc as plsc

_L = 16
_CHUNK = 16
_NCHUNK = 14
_PADLEN = _CHUNK * _NCHUNK
_VOCAB = 100000
_C = 64
_HALF = 32
_SEQLEN = 200
_NKT = _HALF // 8


def _mc_body(seq_hbm, delta_hbm, out0_hbm, out1_hbm,
             seq_v, tiles_v, out_v, tot_v, totbuf_v, shared, sem):
    c = lax.axis_index("c")
    s = lax.axis_index("s")
    active = s < _NCHUNK
    j = s

    @pl.when(active)
    def _gather_and_scan():
        pltpu.sync_copy(seq_hbm, seq_v.at[pl.ds(0, _SEQLEN)])
        sv16 = seq_v[pl.ds(pl.multiple_of(j * _CHUNK, _CHUNK), _CHUNK)]
        sv16 = jnp.clip(sv16, 0, _VOCAB - 1)
        copies = []
        row0 = pl.multiple_of(c * _HALF, 8)
        for p in range(_CHUNK):
            v = sv16[p]
            col0 = pl.multiple_of((v >> 7) * 128, 128)
            copies.append(pltpu.async_copy(
                delta_hbm.at[pl.ds(row0, _HALF), pl.ds(col0, 128)],
                tiles_v.at[pl.ds(p * _HALF, _HALF), :],
                sem,
            ))
        for cp in copies:
            cp.wait()
        lane = lax.iota(jnp.int32, _L)
        acc0 = jnp.zeros((_L,), jnp.float32)
        acc1 = jnp.zeros((_L,), jnp.float32)
        voff16 = sv16 & 127
        for p in range(_CHUNK):
            voff = jnp.full((_L,), voff16[p], jnp.int32)
            r0 = jnp.full((_L,), p * _HALF, jnp.int32) + lane
            acc0 = acc0 + plsc.load_gather(tiles_v, [r0, voff])
            acc1 = acc1 + plsc.load_gather(tiles_v, [r0 + _L, voff])
            out_v[p, pl.ds(0, _L)] = acc0
            out_v[p, pl.ds(_L, _L)] = acc1
        tot_v[pl.ds(0, _L)] = acc0
        tot_v[pl.ds(_L, _L)] = acc1
        pltpu.sync_copy(tot_v, shared.at[c * _L + s, pl.ds(0, _HALF)])

    plsc.subcore_barrier()

    @pl.when(active)
    def _prefix_and_write():
        pltpu.sync_copy(
            shared.at[pl.ds(pl.multiple_of(c * _L, _L), _L), :], totbuf_v)
        off0 = jnp.zeros((_L,), jnp.float32)
        off1 = jnp.zeros((_L,), jnp.float32)
        zero = jnp.zeros((_L,), jnp.float32)
        for i in range(_NCHUNK - 1):
            sel = i < j
            off0 = off0 + jnp.where(sel, totbuf_v[i, pl.ds(0, _L)], zero)
            off1 = off1 + jnp.where(sel, totbuf_v[i, pl.ds(_L, _L)], zero)
        for p in range(_CHUNK):
            out_v[p, pl.ds(0, _L)] = out_v[p, pl.ds(0, _L)] + off0
            out_v[p, pl.ds(_L, _L)] = out_v[p, pl.ds(_L, _L)] + off1

        @pl.when(c == 0)
        def _w0():
            pltpu.sync_copy(out_v, out0_hbm.at[pl.ds(j * _CHUNK, _CHUNK), :])

        @pl.when(c == 1)
        def _w1():
            pltpu.sync_copy(out_v, out1_hbm.at[pl.ds(j * _CHUNK, _CHUNK), :])


@jax.jit
def _mc(seq, delta):
    mesh = plsc.VectorSubcoreMesh(core_axis_name="c", subcore_axis_name="s")
    f = functools.partial(
        pl.kernel,
        out_type=[
            jax.ShapeDtypeStruct((_PADLEN, _HALF), jnp.float32),
            jax.ShapeDtypeStruct((_PADLEN, _HALF), jnp.float32),
        ],
        mesh=mesh,
        compiler_params=pltpu.CompilerParams(
            needs_layout_passes=False, use_tc_tiling_on_sc=True),
        scratch_types=[
            pltpu.VMEM((_PADLEN,), jnp.int32),
            pltpu.VMEM((_CHUNK * _NKT * 8, 128), jnp.float32),
            pltpu.VMEM((_CHUNK, _HALF), jnp.float32),
            pltpu.VMEM((_HALF,), jnp.float32),
            pltpu.VMEM((_L, 128), jnp.float32),
            pltpu.VMEM_SHARED((2 * _L, 128), jnp.float32),
            pltpu.SemaphoreType.DMA,
        ],
    )(_mc_body)
    return f(seq, delta)


def kernel(input_seq, delta):
    out0, out1 = _mc(input_seq.astype(jnp.int32), delta)
    return jnp.concatenate([out0, out1], axis=1)[:_SEQLEN]


# speedup vs baseline: 2.1389x; 1.0109x over previous
import functools

import jax
import jax.numpy as jnp
from jax import lax
from jax.experimental import pallas as pl
from jax.experimental.pallas import tpu as pltpu
from jax.experimental.pallas import tpu_sc as plsc

_L = 16
_CHUNK = 16
_NCHUNK = 13
_PADLEN = _CHUNK * _NCHUNK
_VOCAB = 100000
_C = 64
_HALF = 32
_SEQLEN = 200
_NKT = _HALF // 8


def _mc_body(seq_hbm, delta_hbm, out0_hbm, out1_hbm,
             seq_v, tiles_v, out_v, tot_v, totbuf_v, shared, sem):
    c = lax.axis_index("c")
    s = lax.axis_index("s")
    active = s < _NCHUNK
    j = s

    @pl.when(active)
    def _gather_and_scan():
        @pl.when(j < _NCHUNK - 1)
        def _full():
            pltpu.sync_copy(
                seq_hbm.at[pl.ds(pl.multiple_of(j * _CHUNK, 8), _CHUNK)],
                seq_v)

        @pl.when(j == _NCHUNK - 1)
        def _tail():
            pltpu.sync_copy(
                seq_hbm.at[pl.ds(_SEQLEN - 8, 8)], seq_v.at[pl.ds(0, 8)])

        sv16 = jnp.clip(seq_v[...], 0, _VOCAB - 1)
        copies = []
        row0 = pl.multiple_of(c * _HALF, 8)
        for p in range(_CHUNK):
            v = sv16[p]
            col0 = pl.multiple_of((v >> 7) * 128, 128)
            copies.append(pltpu.async_copy(
                delta_hbm.at[pl.ds(row0, _HALF), pl.ds(col0, 128)],
                tiles_v.at[pl.ds(p * _HALF, _HALF), :],
                sem,
            ))
        for cp in copies:
            cp.wait()
        lane = lax.iota(jnp.int32, _L)
        acc0 = jnp.zeros((_L,), jnp.float32)
        acc1 = jnp.zeros((_L,), jnp.float32)
        voff16 = sv16 & 127
        for p in range(_CHUNK):
            voff = jnp.full((_L,), voff16[p], jnp.int32)
            r0 = jnp.full((_L,), p * _HALF, jnp.int32) + lane
            acc0 = acc0 + plsc.load_gather(tiles_v, [r0, voff])
            acc1 = acc1 + plsc.load_gather(tiles_v, [r0 + _L, voff])
            out_v[p, pl.ds(0, _L)] = acc0
            out_v[p, pl.ds(_L, _L)] = acc1
        tot_v[pl.ds(0, _L)] = acc0
        tot_v[pl.ds(_L, _L)] = acc1
        pltpu.sync_copy(tot_v, shared.at[c * _L + s, pl.ds(0, _HALF)])

    plsc.subcore_barrier()

    @pl.when(active)
    def _prefix_and_write():
        pltpu.sync_copy(
            shared.at[pl.ds(pl.multiple_of(c * _L, _L), _L), :], totbuf_v)
        off0 = jnp.zeros((_L,), jnp.float32)
        off1 = jnp.zeros((_L,), jnp.float32)
        zero = jnp.zeros((_L,), jnp.float32)
        for i in range(_NCHUNK - 1):
            sel = i < j
            off0 = off0 + jnp.where(sel, totbuf_v[i, pl.ds(0, _L)], zero)
            off1 = off1 + jnp.where(sel, totbuf_v[i, pl.ds(_L, _L)], zero)
        for p in range(_CHUNK):
            out_v[p, pl.ds(0, _L)] = out_v[p, pl.ds(0, _L)] + off0
            out_v[p, pl.ds(_L, _L)] = out_v[p, pl.ds(_L, _L)] + off1

        @pl.when(c == 0)
        def _w0():
            pltpu.sync_copy(out_v, out0_hbm.at[pl.ds(j * _CHUNK, _CHUNK), :])

        @pl.when(c == 1)
        def _w1():
            pltpu.sync_copy(out_v, out1_hbm.at[pl.ds(j * _CHUNK, _CHUNK), :])


@jax.jit
def _mc(seq, delta):
    mesh = plsc.VectorSubcoreMesh(core_axis_name="c", subcore_axis_name="s")
    f = functools.partial(
        pl.kernel,
        out_type=[
            jax.ShapeDtypeStruct((_PADLEN, _HALF), jnp.float32),
            jax.ShapeDtypeStruct((_PADLEN, _HALF), jnp.float32),
        ],
        mesh=mesh,
        compiler_params=pltpu.CompilerParams(
            needs_layout_passes=False, use_tc_tiling_on_sc=True),
        scratch_types=[
            pltpu.VMEM((_CHUNK,), jnp.int32),
            pltpu.VMEM((_CHUNK * _NKT * 8, 128), jnp.float32),
            pltpu.VMEM((_CHUNK, _HALF), jnp.float32),
            pltpu.VMEM((_HALF,), jnp.float32),
            pltpu.VMEM((_L, 128), jnp.float32),
            pltpu.VMEM_SHARED((2 * _L, 128), jnp.float32),
            pltpu.SemaphoreType.DMA,
        ],
    )(_mc_body)
    return f(seq, delta)


def kernel(input_seq, delta):
    out0, out1 = _mc(input_seq.astype(jnp.int32), delta)
    return jnp.concatenate([out0[:_SEQLEN], out1[:_SEQLEN]], axis=1)


# speedup vs baseline: 2.1844x; 1.0213x over previous
import functools

import jax
import jax.numpy as jnp
from jax import lax
from jax.experimental import pallas as pl
from jax.experimental.pallas import tpu as pltpu
from jax.experimental.pallas import tpu_sc as plsc

_L = 16
_CHUNK = 16
_NCHUNK = 13
_PADLEN = _CHUNK * _NCHUNK
_VOCAB = 100000
_C = 64
_HALF = 32
_SEQLEN = 200
_NKT = _HALF // 8


def _mc_body(seq_hbm, delta_hbm, out0_hbm, out1_hbm,
             seq_v, tiles_v, out_v, tot_v, totbuf_v, shared, sem):
    c = lax.axis_index("c")
    s = lax.axis_index("s")
    active = s < _NCHUNK
    j = s

    @pl.when(active)
    def _gather_and_scan():
        @pl.when(j < _NCHUNK - 1)
        def _full():
            pltpu.sync_copy(
                seq_hbm.at[pl.ds(pl.multiple_of(j * _CHUNK, 8), _CHUNK)],
                seq_v)

        @pl.when(j == _NCHUNK - 1)
        def _tail():
            pltpu.sync_copy(
                seq_hbm.at[pl.ds(_SEQLEN - 8, 8)], seq_v.at[pl.ds(0, 8)])

        sv16 = jnp.clip(seq_v[...], 0, _VOCAB - 1)
        copies = []
        row0 = pl.multiple_of(c * _HALF, 8)
        for p in range(_CHUNK):
            v = sv16[p]
            col0 = pl.multiple_of((v >> 7) * 128, 128)
            copies.append(pltpu.async_copy(
                delta_hbm.at[pl.ds(row0, _HALF), pl.ds(col0, 128)],
                tiles_v.at[pl.ds(p * _HALF, _HALF), :],
                sem,
            ))
        for cp in copies:
            cp.wait()
        lane = lax.iota(jnp.int32, _L)
        acc0 = jnp.zeros((_L,), jnp.float32)
        acc1 = jnp.zeros((_L,), jnp.float32)
        voff16 = sv16 & 127
        for p in range(_CHUNK):
            voff = jnp.full((_L,), voff16[p], jnp.int32)
            r0 = jnp.full((_L,), p * _HALF, jnp.int32) + lane
            acc0 = acc0 + plsc.load_gather(tiles_v, [r0, voff])
            acc1 = acc1 + plsc.load_gather(tiles_v, [r0 + _L, voff])
            out_v[p, pl.ds(0, _L)] = acc0
            out_v[p, pl.ds(_L, _L)] = acc1
        tot_v[pl.ds(0, _L)] = acc0
        tot_v[pl.ds(_L, _L)] = acc1
        pltpu.sync_copy(tot_v, shared.at[c * _L + s, pl.ds(0, _HALF)])

    plsc.subcore_barrier()

    @pl.when(active)
    def _prefix_and_write():
        pltpu.sync_copy(
            shared.at[pl.ds(pl.multiple_of(c * _L, _L), _L), :], totbuf_v)
        off0 = jnp.zeros((_L,), jnp.float32)
        off1 = jnp.zeros((_L,), jnp.float32)
        zero = jnp.zeros((_L,), jnp.float32)
        for i in range(_NCHUNK - 1):
            sel = i < j
            off0 = off0 + jnp.where(sel, totbuf_v[i, pl.ds(0, _L)], zero)
            off1 = off1 + jnp.where(sel, totbuf_v[i, pl.ds(_L, _L)], zero)
        for p in range(_CHUNK):
            out_v[p, pl.ds(0, _L)] = out_v[p, pl.ds(0, _L)] + off0
            out_v[p, pl.ds(_L, _L)] = out_v[p, pl.ds(_L, _L)] + off1

        last = j == _NCHUNK - 1
        for cc, dst in ((0, out0_hbm), (1, out1_hbm)):
            @pl.when(jnp.logical_and(c == cc, jnp.logical_not(last)))
            def _w(dst=dst):
                pltpu.sync_copy(
                    out_v, dst.at[pl.ds(j * _CHUNK, _CHUNK), :])

            @pl.when(jnp.logical_and(c == cc, last))
            def _wl(dst=dst):
                pltpu.sync_copy(
                    out_v.at[pl.ds(0, 8), :],
                    dst.at[pl.ds(_SEQLEN - 8, 8), :])


@jax.jit
def _mc(seq, delta):
    mesh = plsc.VectorSubcoreMesh(core_axis_name="c", subcore_axis_name="s")
    f = functools.partial(
        pl.kernel,
        out_type=[
            jax.ShapeDtypeStruct((_SEQLEN, _HALF), jnp.float32),
            jax.ShapeDtypeStruct((_SEQLEN, _HALF), jnp.float32),
        ],
        mesh=mesh,
        compiler_params=pltpu.CompilerParams(
            needs_layout_passes=False, use_tc_tiling_on_sc=True),
        scratch_types=[
            pltpu.VMEM((_CHUNK,), jnp.int32),
            pltpu.VMEM((_CHUNK * _NKT * 8, 128), jnp.float32),
            pltpu.VMEM((_CHUNK, _HALF), jnp.float32),
            pltpu.VMEM((_HALF,), jnp.float32),
            pltpu.VMEM((_L, 128), jnp.float32),
            pltpu.VMEM_SHARED((2 * _L, 128), jnp.float32),
            pltpu.SemaphoreType.DMA,
        ],
    )(_mc_body)
    return f(seq, delta)


def kernel(input_seq, delta):
    out0, out1 = _mc(input_seq.astype(jnp.int32), delta)
    return jnp.concatenate([out0, out1], axis=1)


# speedup vs baseline: 2.1921x; 1.0035x over previous
import functools

import jax
import jax.numpy as jnp
from jax import lax
from jax.experimental import pallas as pl
from jax.experimental.pallas import tpu as pltpu
from jax.experimental.pallas import tpu_sc as plsc

_L = 16
_CHUNK = 16
_NCHUNK = 13
_PADLEN = _CHUNK * _NCHUNK
_VOCAB = 100000
_C = 64
_HALF = 32
_SEQLEN = 200
_NKT = _HALF // 8


def _mc_body(seq_hbm, delta_hbm, out0_hbm, out1_hbm,
             seq_v, tiles_v, out_v, wide_v, tot_v, totbuf_v, shared, sem):
    c = lax.axis_index("c")
    s = lax.axis_index("s")
    active = s < _NCHUNK
    j = s

    @pl.when(active)
    def _gather_and_scan():
        @pl.when(j < _NCHUNK - 1)
        def _full():
            pltpu.sync_copy(
                seq_hbm.at[pl.ds(pl.multiple_of(j * _CHUNK, 8), _CHUNK)],
                seq_v)

        @pl.when(j == _NCHUNK - 1)
        def _tail():
            pltpu.sync_copy(
                seq_hbm.at[pl.ds(_SEQLEN - 8, 8)], seq_v.at[pl.ds(0, 8)])

        sv16 = jnp.clip(seq_v[...], 0, _VOCAB - 1)
        copies = []
        row0 = pl.multiple_of(c * _HALF, 8)
        for p in range(_CHUNK):
            v = sv16[p]
            col0 = pl.multiple_of((v >> 7) * 128, 128)
            copies.append(pltpu.async_copy(
                delta_hbm.at[pl.ds(row0, _HALF), pl.ds(col0, 128)],
                tiles_v.at[pl.ds(p * _HALF, _HALF), :],
                sem,
            ))
        for cp in copies:
            cp.wait()
        lane = lax.iota(jnp.int32, _L)
        acc0 = jnp.zeros((_L,), jnp.float32)
        acc1 = jnp.zeros((_L,), jnp.float32)
        voff16 = sv16 & 127
        for p in range(_CHUNK):
            voff = jnp.full((_L,), voff16[p], jnp.int32)
            r0 = jnp.full((_L,), p * _HALF, jnp.int32) + lane
            acc0 = acc0 + plsc.load_gather(tiles_v, [r0, voff])
            acc1 = acc1 + plsc.load_gather(tiles_v, [r0 + _L, voff])
            out_v[p, pl.ds(0, _L)] = acc0
            out_v[p, pl.ds(_L, _L)] = acc1
        tot_v[pl.ds(0, _L)] = acc0
        tot_v[pl.ds(_L, _L)] = acc1
        pltpu.sync_copy(tot_v, shared.at[c * _L + s, pl.ds(0, _HALF)])

    plsc.subcore_barrier()

    @pl.when(active)
    def _prefix_and_write():
        pltpu.sync_copy(
            shared.at[pl.ds(pl.multiple_of(c * _L, _L), _L), :], totbuf_v)
        off0 = jnp.zeros((_L,), jnp.float32)
        off1 = jnp.zeros((_L,), jnp.float32)
        zero = jnp.zeros((_L,), jnp.float32)
        for i in range(_NCHUNK - 1):
            sel = i < j
            off0 = off0 + jnp.where(sel, totbuf_v[i, pl.ds(0, _L)], zero)
            off1 = off1 + jnp.where(sel, totbuf_v[i, pl.ds(_L, _L)], zero)
        c0 = c == 0
        for p in range(_CHUNK):
            h0 = out_v[p, pl.ds(0, _L)] + off0
            h1 = out_v[p, pl.ds(_L, _L)] + off1
            wide_v[p, pl.ds(0, _L)] = jnp.where(c0, h0, zero)
            wide_v[p, pl.ds(_L, _L)] = jnp.where(c0, h1, zero)
            wide_v[p, pl.ds(2 * _L, _L)] = jnp.where(c0, zero, h0)
            wide_v[p, pl.ds(3 * _L, _L)] = jnp.where(c0, zero, h1)

        last = j == _NCHUNK - 1
        for cc, dst in ((0, out0_hbm), (1, out1_hbm)):
            @pl.when(jnp.logical_and(c == cc, jnp.logical_not(last)))
            def _w(dst=dst):
                pltpu.sync_copy(
                    wide_v, dst.at[pl.ds(j * _CHUNK, _CHUNK), :])

            @pl.when(jnp.logical_and(c == cc, last))
            def _wl(dst=dst):
                pltpu.sync_copy(
                    wide_v.at[pl.ds(0, 8), :],
                    dst.at[pl.ds(_SEQLEN - 8, 8), :])


@jax.jit
def _mc(seq, delta):
    mesh = plsc.VectorSubcoreMesh(core_axis_name="c", subcore_axis_name="s")
    f = functools.partial(
        pl.kernel,
        out_type=[
            jax.ShapeDtypeStruct((_SEQLEN, _C), jnp.float32),
            jax.ShapeDtypeStruct((_SEQLEN, _C), jnp.float32),
        ],
        mesh=mesh,
        compiler_params=pltpu.CompilerParams(
            needs_layout_passes=False, use_tc_tiling_on_sc=True),
        scratch_types=[
            pltpu.VMEM((_CHUNK,), jnp.int32),
            pltpu.VMEM((_CHUNK * _NKT * 8, 128), jnp.float32),
            pltpu.VMEM((_CHUNK, _HALF), jnp.float32),
            pltpu.VMEM((_CHUNK, _C), jnp.float32),
            pltpu.VMEM((_HALF,), jnp.float32),
            pltpu.VMEM((_L, 128), jnp.float32),
            pltpu.VMEM_SHARED((2 * _L, 128), jnp.float32),
            pltpu.SemaphoreType.DMA,
        ],
    )(_mc_body)
    return f(seq, delta)


def kernel(input_seq, delta):
    out0, out1 = _mc(input_seq.astype(jnp.int32), delta)
    return out0 + out1
